# baseline (speedup 1.0000x reference)
import functools

import jax
import jax.numpy as jnp
from jax import lax
from jax.experimental import pallas as pl
from jax.experimental.pallas import tpu as pltpu
from jax.experimental.pallas import tpu_sc as plsc

_W0 = 1.0 / 1223
_W1 = 1.0 / 2444
_W2 = 1.0 / 1687
_LN2 = 0.6931471805599453
_C0 = -1.14299441
_C1 = 1.38276158
_C2 = -0.2335087

_B = 16384
_NC = 2
_NS = 16
_NW = _NC * _NS
_PER = _B // _NW
_NV = _PER // 16
_NIDX = 12


def _sc_loss(x_hbm, t_hbm, stage_hbm, out_hbm, idxv, xv, tv, accv,
             redv, sem):
    sub = lax.axis_index("s")
    core = lax.axis_index("c")
    wid = core * _NS + sub
    base = wid * _PER
    lane = lax.broadcasted_iota(jnp.int32, (16,), 0)
    for i in range(_NIDX):
        cls = i // 4
        blk = i % 4
        for p in range(8):
            idxv[i, pl.ds(p * 16, 16)] = (
                lane + (blk * 128 + p * 16) + base) * 3 + cls
    pltpu.sync_copy(t_hbm.at[pl.ds(base, _PER)], tv)
    copies = [
        pltpu.async_copy(x_hbm.at[idxv.at[i]],
                         xv.at[pl.ds(i * 128, 128)], sem)
        for i in range(_NIDX)
    ]
    for c in copies:
        c.wait()
    accn = jnp.zeros((16,), jnp.float32)
    accd = jnp.zeros((16,), jnp.float32)
    for j in range(_NV):
        a0 = xv[pl.ds(j * 16, 16)]
        a1 = xv[pl.ds(512 + j * 16, 16)]
        a2 = xv[pl.ds(1024 + j * 16, 16)]
        tl = tv[pl.ds(j * 16, 16)]
        s = jnp.exp(a0) + jnp.exp(a1) + jnp.exp(a2)
        bits = lax.bitcast_convert_type(s, jnp.int32)
        ex = ((bits >> 23) & 0xFF) - 127
        mant = lax.bitcast_convert_type(
            (bits & 0x7FFFFF) | 0x3F800000, jnp.float32)
        y = ex.astype(jnp.float32) * _LN2 + (_C0 + mant * (_C1 + mant * _C2))
        y = y - 1.0 + s * jnp.exp(-y)
        lse = y - 1.0 + s * jnp.exp(-y)
        is0 = tl == 0
        is1 = tl == 1
        picked = jnp.where(is0, a0, jnp.where(is1, a1, a2))
        wv = jnp.where(is0, _W0, jnp.where(is1, _W1, _W2))
        accn = accn + wv * (lse - picked)
        accd = accd + wv
    accv[pl.ds(0, 16)] = accn
    accv[pl.ds(16, 16)] = accd
    pltpu.sync_copy(accv.at[pl.ds(0, 16)], stage_hbm.at[pl.ds(wid * 16, 16)])
    pltpu.sync_copy(accv.at[pl.ds(16, 16)],
                    stage_hbm.at[pl.ds(512 + wid * 16, 16)])
    plsc.subcore_barrier()

    @pl.when(sub == 0)
    def _():
        pltpu.sync_copy(stage_hbm.at[pl.ds(core * 256, 256)],
                        redv.at[pl.ds(0, 256)])
        pltpu.sync_copy(stage_hbm.at[pl.ds(512 + core * 256, 256)],
                        redv.at[pl.ds(256, 256)])
        tn = jnp.zeros((16,), jnp.float32)
        td = jnp.zeros((16,), jnp.float32)
        for r in range(_NS):
            tn = tn + redv[pl.ds(r * 16, 16)]
            td = td + redv[pl.ds(256 + r * 16, 16)]
        sn = jnp.float32(0.0)
        sd = jnp.float32(0.0)
        for k in range(16):
            sn = sn + tn[k]
            sd = sd + td[k]
        snv = jnp.full((16,), sn, jnp.float32)
        sdv = jnp.full((16,), sd, jnp.float32)
        accv[pl.ds(0, 16)] = jnp.where(lane == 0, snv,
                                       jnp.where(lane == 1, sdv, 0.0))
        pltpu.sync_copy(accv.at[pl.ds(0, 16)],
                        out_hbm.at[pl.ds(core * 16, 16)])


def kernel(index, output, target, pred_hist):
    del index, pred_hist
    x = output.reshape(_B * 3)
    mesh = plsc.VectorSubcoreMesh(core_axis_name="c", subcore_axis_name="s")
    _, part = functools.partial(
        pl.kernel,
        mesh=mesh,
        out_type=(
            jax.ShapeDtypeStruct((1024,), jnp.float32),
            jax.ShapeDtypeStruct((32,), jnp.float32),
        ),
        scratch_types=[
            pltpu.VMEM((_NIDX, 128), jnp.int32),
            pltpu.VMEM((_PER * 3,), jnp.float32),
            pltpu.VMEM((_PER,), jnp.int32),
            pltpu.VMEM((32,), jnp.float32),
            pltpu.VMEM((512,), jnp.float32),
            pltpu.SemaphoreType.DMA,
        ],
    )(_sc_loss)(x, target)
    return (part[0] + part[16]) / (part[1] + part[17])


# speedup vs baseline: 19.8593x; 19.8593x over previous
import jax
import jax.numpy as jnp
from jax.experimental import pallas as pl
from jax.experimental.pallas import tpu as pltpu

_W0 = 1.0 / 1223
_W1 = 1.0 / 2444
_W2 = 1.0 / 1687


def _ce_kernel(x_ref, t_ref, loss_ref):
    x = x_ref[...]
    t = t_ref[...]
    e = jnp.exp(x)
    lse = jnp.log(e[0:1, :] + e[1:2, :] + e[2:3, :])
    is0 = t == 0
    is1 = t == 1
    picked = jnp.where(is0, x[0:1, :], jnp.where(is1, x[1:2, :], x[2:3, :]))
    w = jnp.where(is0, _W0, jnp.where(is1, _W1, _W2)).astype(jnp.float32)
    num = jnp.sum(w * (lse - picked))
    den = jnp.sum(w)
    loss_ref[0, 0] = num / den


def kernel(index, output, target, pred_hist):
    del index, pred_hist
    x = output.T
    t = target.reshape(1, 16384)
    loss = pl.pallas_call(
        _ce_kernel,
        out_shape=jax.ShapeDtypeStruct((1, 1), jnp.float32),
        out_specs=pl.BlockSpec(memory_space=pltpu.SMEM),
    )(x, t)
    return loss[0, 0]
